# Initial kernel scaffold; baseline (speedup 1.0000x reference)
import jax
import jax.numpy as jnp
from jax.experimental import pallas as pl


def kernel(edge_index, estimates, W):
    raise NotImplementedError("write your pallas kernel here")


# speedup vs baseline: 4.0911x; 4.0911x over previous
import functools

import jax
import jax.numpy as jnp
from jax import lax
from jax.experimental import pallas as pl
from jax.experimental.pallas import tpu as pltpu
from jax.experimental.pallas import tpu_sc as plsc

NUM_ITERS = 10
ALPHA = 0.1

NS = 16
CHUNK = 128


GROUP = 32


def _make_scatter_kernel(n_chunks_t, pn, c):
    rows_per_tile = pn // NS
    n_zero = rows_per_tile // CHUNK
    n_groups = n_chunks_t // GROUP

    @functools.partial(
        pl.kernel,
        out_type=jax.ShapeDtypeStruct((pn, c), jnp.float32),
        mesh=plsc.VectorSubcoreMesh(core_axis_name="c", subcore_axis_name="s",
                                    num_cores=1),
        scratch_types=[
            pltpu.VMEM((GROUP, CHUNK), jnp.int32),
            pltpu.VMEM((GROUP, CHUNK), jnp.int32),
            pltpu.VMEM((CHUNK, c), jnp.float32),
            pltpu.VMEM_SHARED((pn, c), jnp.float32),
            pltpu.SemaphoreType.DMA,
        ],
    )
    def scatter_kernel(src_hbm, dst_hbm, y_hbm, zeros_hbm, out_hbm,
                       src_idx, dst_idx, rowbuf, acc, sem):
        sid = lax.axis_index("s")
        base = sid * n_chunks_t

        pltpu.sync_copy(zeros_hbm, rowbuf)

        def zbody(r, carry):
            pltpu.sync_copy(
                rowbuf, acc.at[pl.ds(sid * rows_per_tile + r * CHUNK, CHUNK)])
            return carry

        lax.fori_loop(0, n_zero, zbody, 0, unroll=False)
        plsc.subcore_barrier()

        def group(g, carry):
            pltpu.sync_copy(src_hbm.at[pl.ds(base + g * GROUP, GROUP)], src_idx)
            pltpu.sync_copy(dst_hbm.at[pl.ds(base + g * GROUP, GROUP)], dst_idx)

            def body(j, carry2):
                pltpu.async_copy(y_hbm.at[src_idx.at[j]], rowbuf, sem).wait()
                pltpu.sync_copy(rowbuf, acc.at[dst_idx.at[j]], add=True)
                return carry2

            lax.fori_loop(0, GROUP, body, 0, unroll=False)
            return carry

        lax.fori_loop(0, n_groups, group, 0, unroll=False)
        plsc.subcore_barrier()

        def obody(r, carry):
            off = sid * rows_per_tile + r * CHUNK
            pltpu.sync_copy(acc.at[pl.ds(off, CHUNK)],
                            out_hbm.at[pl.ds(off, CHUNK)])
            return carry

        lax.fori_loop(0, n_zero, obody, 0, unroll=False)

    return scatter_kernel


def _prep_body(w_ref, deg_ref, p_ref, norm_ref):
    w = w_ref[...]
    m = jnp.max(w, axis=1, keepdims=True)
    e = jnp.exp(w - m)
    p_ref[...] = e / jnp.sum(e, axis=1, keepdims=True)
    norm_ref[...] = 1.0 / jnp.maximum(deg_ref[...], 1.0)


def _y0_body(est_ref, p_ref, y_ref):
    y_ref[...] = jnp.dot(est_ref[...], p_ref[...],
                         preferred_element_type=jnp.float32)


def _blend_body(parts_ref, norm_ref, est0_ref, p_ref, blend_ref, y_ref):
    b = ((1.0 - ALPHA) * parts_ref[...] * norm_ref[...]
         + ALPHA * est0_ref[...])
    blend_ref[...] = b
    y_ref[...] = jnp.dot(b, p_ref[...], preferred_element_type=jnp.float32)


def kernel(edge_index, estimates, W):
    n, c = estimates.shape
    e = edge_index.shape[1]

    n_chunks_t = -(-(-(-e // (NS * CHUNK))) // 8) * 8
    e_pad = n_chunks_t * CHUNK * NS
    pn = -(-n // (NS * CHUNK)) * (NS * CHUNK)
    pad = e_pad - e
    pad_ar = jnp.arange(pad, dtype=jnp.int32)
    src = jnp.concatenate([edge_index[0], pad_ar % n]).reshape(-1, CHUNK)
    dst = jnp.concatenate([edge_index[1], n + pad_ar % (pn - n)]).reshape(-1, CHUNK)

    zeros_c = jnp.zeros((CHUNK, c), jnp.float32)
    ones_t = jnp.ones((n, c), jnp.float32)

    scatter = _make_scatter_kernel(n_chunks_t, pn, c)

    deg = scatter(src, dst, ones_t, zeros_c)

    p_mat, norm = pl.pallas_call(
        _prep_body,
        out_shape=(jax.ShapeDtypeStruct((c, c), jnp.float32),
                   jax.ShapeDtypeStruct((pn, c), jnp.float32)),
    )(W, deg)

    bn = 2000
    grid = n // bn
    y = pl.pallas_call(
        _y0_body,
        grid=(grid,),
        in_specs=[pl.BlockSpec((bn, c), lambda i: (i, 0)),
                  pl.BlockSpec((c, c), lambda i: (0, 0))],
        out_specs=pl.BlockSpec((bn, c), lambda i: (i, 0)),
        out_shape=jax.ShapeDtypeStruct((n, c), jnp.float32),
    )(estimates, p_mat)

    blend_call = pl.pallas_call(
        _blend_body,
        grid=(grid,),
        in_specs=[pl.BlockSpec((bn, c), lambda i: (i, 0)),
                  pl.BlockSpec((bn, c), lambda i: (i, 0)),
                  pl.BlockSpec((bn, c), lambda i: (i, 0)),
                  pl.BlockSpec((c, c), lambda i: (0, 0))],
        out_specs=(pl.BlockSpec((bn, c), lambda i: (i, 0)),
                   pl.BlockSpec((bn, c), lambda i: (i, 0))),
        out_shape=(jax.ShapeDtypeStruct((n, c), jnp.float32),
                   jax.ShapeDtypeStruct((n, c), jnp.float32)),
    )

    est = estimates
    for _ in range(NUM_ITERS):
        parts = scatter(src, dst, y, zeros_c)
        est, y = blend_call(parts, norm, estimates, p_mat)
    return est


# speedup vs baseline: 5.6175x; 1.3731x over previous
import functools

import jax
import jax.numpy as jnp
from jax import lax
from jax.experimental import pallas as pl
from jax.experimental.pallas import tpu as pltpu
from jax.experimental.pallas import tpu_sc as plsc

NUM_ITERS = 10
ALPHA = 0.1

NS = 16
CHUNK = 128


GROUP = 40


def _make_scatter_kernel(n_chunks_t, pn, c):
    rows_per_tile = pn // NS
    n_zero = rows_per_tile // CHUNK
    n_groups = n_chunks_t // GROUP
    n_pairs = GROUP // 2

    @functools.partial(
        pl.kernel,
        out_type=jax.ShapeDtypeStruct((pn, c), jnp.float32),
        mesh=plsc.VectorSubcoreMesh(core_axis_name="c", subcore_axis_name="s",
                                    num_cores=1),
        scratch_types=[
            pltpu.VMEM((GROUP, CHUNK), jnp.int32),
            pltpu.VMEM((GROUP, CHUNK), jnp.int32),
            pltpu.VMEM((CHUNK, c), jnp.float32),
            pltpu.VMEM((CHUNK, c), jnp.float32),
            pltpu.VMEM_SHARED((pn, c), jnp.float32),
            pltpu.SemaphoreType.DMA,
            pltpu.SemaphoreType.DMA,
            pltpu.SemaphoreType.DMA,
            pltpu.SemaphoreType.DMA,
        ],
    )
    def scatter_kernel(src_hbm, dst_hbm, y_hbm, zeros_hbm, out_hbm,
                       src_idx, dst_idx, buf0, buf1, acc,
                       semg0, semg1, sems0, sems1):
        sid = lax.axis_index("s")
        base = sid * n_chunks_t

        pltpu.sync_copy(zeros_hbm, buf0)

        def zbody(r, carry):
            pltpu.sync_copy(
                buf0, acc.at[pl.ds(sid * rows_per_tile + r * CHUNK, CHUNK)])
            return carry

        lax.fori_loop(0, n_zero, zbody, 0, unroll=False)
        plsc.subcore_barrier()

        def gather(j, buf, sem):
            return pltpu.async_copy(y_hbm.at[src_idx.at[j]], buf, sem)

        def scatter(j, buf, sem):
            return pltpu.async_copy(buf, acc.at[dst_idx.at[j]], sem, add=True)

        def group(g, carry):
            pltpu.sync_copy(src_hbm.at[pl.ds(base + g * GROUP, GROUP)], src_idx)
            pltpu.sync_copy(dst_hbm.at[pl.ds(base + g * GROUP, GROUP)], dst_idx)
            gather(0, buf0, semg0)

            def pair(p, carry2):
                j0 = 2 * p
                j1 = j0 + 1
                pltpu.make_async_copy(y_hbm.at[src_idx.at[j0]], buf0,
                                      semg0).wait()
                scatter(j0, buf0, sems0)

                @pl.when(p > 0)
                def _():
                    pltpu.make_async_copy(
                        buf1, acc.at[dst_idx.at[j1 - 2]], sems1).wait()

                gather(j1, buf1, semg1)
                pltpu.make_async_copy(y_hbm.at[src_idx.at[j1]], buf1,
                                      semg1).wait()
                pltpu.make_async_copy(buf0, acc.at[dst_idx.at[j0]],
                                      sems0).wait()
                scatter(j1, buf1, sems1)

                @pl.when(p < n_pairs - 1)
                def _():
                    gather(j0 + 2, buf0, semg0)

                return carry2

            lax.fori_loop(0, n_pairs, pair, 0, unroll=False)
            pltpu.make_async_copy(buf1, acc.at[dst_idx.at[GROUP - 1]],
                                  sems1).wait()
            return carry

        lax.fori_loop(0, n_groups, group, 0, unroll=False)
        plsc.subcore_barrier()

        def obody(r, carry):
            off = sid * rows_per_tile + r * CHUNK
            pltpu.sync_copy(acc.at[pl.ds(off, CHUNK)],
                            out_hbm.at[pl.ds(off, CHUNK)])
            return carry

        lax.fori_loop(0, n_zero, obody, 0, unroll=False)

    return scatter_kernel


def _prep_body(w_ref, deg_ref, p_ref, norm_ref):
    w = w_ref[...]
    m = jnp.max(w, axis=1, keepdims=True)
    e = jnp.exp(w - m)
    p_ref[...] = e / jnp.sum(e, axis=1, keepdims=True)
    norm_ref[...] = 1.0 / jnp.maximum(deg_ref[...], 1.0)


def _y0_body(est_ref, p_ref, y_ref):
    y_ref[...] = jnp.dot(est_ref[...], p_ref[...],
                         preferred_element_type=jnp.float32)


def _blend_body(parts_ref, norm_ref, est0_ref, p_ref, blend_ref, y_ref):
    b = ((1.0 - ALPHA) * parts_ref[...] * norm_ref[...]
         + ALPHA * est0_ref[...])
    blend_ref[...] = b
    y_ref[...] = jnp.dot(b, p_ref[...], preferred_element_type=jnp.float32)


def kernel(edge_index, estimates, W):
    n, c = estimates.shape
    e = edge_index.shape[1]

    n_chunks_t = -(-(-(-e // (NS * CHUNK))) // 8) * 8
    e_pad = n_chunks_t * CHUNK * NS
    pn = -(-n // (NS * CHUNK)) * (NS * CHUNK)
    pad = e_pad - e
    pad_ar = jnp.arange(pad, dtype=jnp.int32)
    src = jnp.concatenate([edge_index[0], pad_ar % n]).reshape(-1, CHUNK)
    dst = jnp.concatenate([edge_index[1], n + pad_ar % (pn - n)]).reshape(-1, CHUNK)

    zeros_c = jnp.zeros((CHUNK, c), jnp.float32)
    ones_t = jnp.ones((n, c), jnp.float32)

    scatter = _make_scatter_kernel(n_chunks_t, pn, c)

    deg = scatter(src, dst, ones_t, zeros_c)

    p_mat, norm = pl.pallas_call(
        _prep_body,
        out_shape=(jax.ShapeDtypeStruct((c, c), jnp.float32),
                   jax.ShapeDtypeStruct((pn, c), jnp.float32)),
    )(W, deg)

    bn = 2000
    grid = n // bn
    y = pl.pallas_call(
        _y0_body,
        grid=(grid,),
        in_specs=[pl.BlockSpec((bn, c), lambda i: (i, 0)),
                  pl.BlockSpec((c, c), lambda i: (0, 0))],
        out_specs=pl.BlockSpec((bn, c), lambda i: (i, 0)),
        out_shape=jax.ShapeDtypeStruct((n, c), jnp.float32),
    )(estimates, p_mat)

    blend_call = pl.pallas_call(
        _blend_body,
        grid=(grid,),
        in_specs=[pl.BlockSpec((bn, c), lambda i: (i, 0)),
                  pl.BlockSpec((bn, c), lambda i: (i, 0)),
                  pl.BlockSpec((bn, c), lambda i: (i, 0)),
                  pl.BlockSpec((c, c), lambda i: (0, 0))],
        out_specs=(pl.BlockSpec((bn, c), lambda i: (i, 0)),
                   pl.BlockSpec((bn, c), lambda i: (i, 0))),
        out_shape=(jax.ShapeDtypeStruct((n, c), jnp.float32),
                   jax.ShapeDtypeStruct((n, c), jnp.float32)),
    )

    est = estimates
    for _ in range(NUM_ITERS):
        parts = scatter(src, dst, y, zeros_c)
        est, y = blend_call(parts, norm, estimates, p_mat)
    return est


# speedup vs baseline: 7.3047x; 1.3003x over previous
import functools

import jax
import jax.numpy as jnp
from jax import lax
from jax.experimental import pallas as pl
from jax.experimental.pallas import tpu as pltpu
from jax.experimental.pallas import tpu_sc as plsc

NUM_ITERS = 10
ALPHA = 0.1

NC = 2
NS = 16
CHUNK = 128
GROUP = 40


def _make_scatter_kernel(n_chunks_t, pn, ch):
    rows_per_tile = pn // NS
    n_zero = rows_per_tile // CHUNK
    n_groups = n_chunks_t // GROUP
    n_pairs = GROUP // 2

    @functools.partial(
        pl.kernel,
        out_type=jax.ShapeDtypeStruct((NC, pn, ch), jnp.float32),
        mesh=plsc.VectorSubcoreMesh(core_axis_name="c", subcore_axis_name="s",
                                    num_cores=NC),
        compiler_params=pltpu.CompilerParams(use_tc_tiling_on_sc=False),
        scratch_types=[
            pltpu.VMEM((GROUP, CHUNK), jnp.int32),
            pltpu.VMEM((GROUP, CHUNK), jnp.int32),
            pltpu.VMEM((CHUNK, ch), jnp.float32),
            pltpu.VMEM((CHUNK, ch), jnp.float32),
            pltpu.VMEM_SHARED((pn, ch), jnp.float32),
            pltpu.SemaphoreType.DMA,
            pltpu.SemaphoreType.DMA,
            pltpu.SemaphoreType.DMA,
            pltpu.SemaphoreType.DMA,
        ],
    )
    def scatter_kernel(src2_hbm, dst_hbm, y2_hbm, zeros_hbm, out_hbm,
                       src_idx, dst_idx, buf0, buf1, acc,
                       semg0, semg1, sems0, sems1):
        cid = lax.axis_index("c")
        sid = lax.axis_index("s")
        base = sid * n_chunks_t

        pltpu.sync_copy(zeros_hbm, buf0)

        def zbody(r, carry):
            pltpu.sync_copy(
                buf0, acc.at[pl.ds(sid * rows_per_tile + r * CHUNK, CHUNK)])
            return carry

        lax.fori_loop(0, n_zero, zbody, 0, unroll=False)
        plsc.subcore_barrier()

        def gather(j, buf, sem):
            return pltpu.async_copy(y2_hbm.at[src_idx.at[j]], buf, sem)

        def scatter(j, buf, sem):
            return pltpu.async_copy(buf, acc.at[dst_idx.at[j]], sem, add=True)

        def group(g, carry):
            pltpu.sync_copy(
                src2_hbm.at[cid, pl.ds(base + g * GROUP, GROUP)], src_idx)
            pltpu.sync_copy(
                dst_hbm.at[pl.ds(base + g * GROUP, GROUP)], dst_idx)
            gather(0, buf0, semg0)

            def pair(p, carry2):
                j0 = 2 * p
                j1 = j0 + 1
                pltpu.make_async_copy(y2_hbm.at[src_idx.at[j0]], buf0,
                                      semg0).wait()
                scatter(j0, buf0, sems0)

                @pl.when(p > 0)
                def _():
                    pltpu.make_async_copy(
                        buf1, acc.at[dst_idx.at[j1 - 2]], sems1).wait()

                gather(j1, buf1, semg1)
                pltpu.make_async_copy(y2_hbm.at[src_idx.at[j1]], buf1,
                                      semg1).wait()
                pltpu.make_async_copy(buf0, acc.at[dst_idx.at[j0]],
                                      sems0).wait()
                scatter(j1, buf1, sems1)

                @pl.when(p < n_pairs - 1)
                def _():
                    gather(j0 + 2, buf0, semg0)

                return carry2

            lax.fori_loop(0, n_pairs, pair, 0, unroll=False)
            pltpu.make_async_copy(buf1, acc.at[dst_idx.at[GROUP - 1]],
                                  sems1).wait()
            return carry

        lax.fori_loop(0, n_groups, group, 0, unroll=False)
        plsc.subcore_barrier()

        def obody(r, carry):
            off = sid * rows_per_tile + r * CHUNK
            pltpu.sync_copy(acc.at[pl.ds(off, CHUNK)],
                            out_hbm.at[cid, pl.ds(off, CHUNK)])
            return carry

        lax.fori_loop(0, n_zero, obody, 0, unroll=False)

    return scatter_kernel


def _prep_body(w_ref, degp_ref, p_ref, norm_ref):
    w = w_ref[...]
    m = jnp.max(w, axis=1, keepdims=True)
    e = jnp.exp(w - m)
    p_ref[...] = e / jnp.sum(e, axis=1, keepdims=True)
    nrm = 1.0 / jnp.maximum(degp_ref[0], 1.0)
    norm_ref[...] = jnp.concatenate([nrm, nrm], axis=1)


def _y0_body(est_ref, p_ref, y_ref):
    y_ref[...] = jnp.dot(est_ref[...], p_ref[...],
                         preferred_element_type=jnp.float32)


def _blend_body(parts_ref, norm_ref, est0_ref, p_ref, blend_ref, y_ref):
    s = jnp.concatenate([parts_ref[0], parts_ref[1]], axis=1)
    b = (1.0 - ALPHA) * s * norm_ref[...] + ALPHA * est0_ref[...]
    blend_ref[...] = b
    y_ref[...] = jnp.dot(b, p_ref[...], preferred_element_type=jnp.float32)


def kernel(edge_index, estimates, W):
    n, c = estimates.shape
    ch = c // 2
    e = edge_index.shape[1]

    n_chunks_t = -(-(-(-e // (NS * CHUNK))) // GROUP) * GROUP
    e_pad = n_chunks_t * CHUNK * NS
    pn = -(-n // (NS * CHUNK)) * (NS * CHUNK)
    pad = e_pad - e
    pad_ar = jnp.arange(pad, dtype=jnp.int32)
    src = jnp.concatenate([edge_index[0], pad_ar % n]).reshape(-1, CHUNK)
    dst = jnp.concatenate([edge_index[1], n + pad_ar % (pn - n)]).reshape(-1, CHUNK)
    src2 = jnp.stack([2 * src, 2 * src + 1])

    zeros_h = jnp.zeros((CHUNK, ch), jnp.float32)
    ones_t = jnp.ones((2 * n, ch), jnp.float32)

    scatter = _make_scatter_kernel(n_chunks_t, pn, ch)

    deg_parts = scatter(src2, dst, ones_t, zeros_h)

    p_mat, norm = pl.pallas_call(
        _prep_body,
        out_shape=(jax.ShapeDtypeStruct((c, c), jnp.float32),
                   jax.ShapeDtypeStruct((pn, c), jnp.float32)),
    )(W, deg_parts)

    bn = 2000
    grid = n // bn
    y = pl.pallas_call(
        _y0_body,
        grid=(grid,),
        in_specs=[pl.BlockSpec((bn, c), lambda i: (i, 0)),
                  pl.BlockSpec((c, c), lambda i: (0, 0))],
        out_specs=pl.BlockSpec((bn, c), lambda i: (i, 0)),
        out_shape=jax.ShapeDtypeStruct((n, c), jnp.float32),
    )(estimates, p_mat)

    blend_call = pl.pallas_call(
        _blend_body,
        grid=(grid,),
        in_specs=[pl.BlockSpec((NC, bn, ch), lambda i: (0, i, 0)),
                  pl.BlockSpec((bn, c), lambda i: (i, 0)),
                  pl.BlockSpec((bn, c), lambda i: (i, 0)),
                  pl.BlockSpec((c, c), lambda i: (0, 0))],
        out_specs=(pl.BlockSpec((bn, c), lambda i: (i, 0)),
                   pl.BlockSpec((bn, c), lambda i: (i, 0))),
        out_shape=(jax.ShapeDtypeStruct((n, c), jnp.float32),
                   jax.ShapeDtypeStruct((n, c), jnp.float32)),
    )

    est = estimates
    for _ in range(NUM_ITERS):
        parts = scatter(src2, dst, y.reshape(2 * n, ch), zeros_h)
        est, y = blend_call(parts, norm, estimates, p_mat)
    return est


# speedup vs baseline: 10.8356x; 1.4834x over previous
import functools

import jax
import jax.numpy as jnp
from jax import lax
from jax.experimental import pallas as pl
from jax.experimental.pallas import tpu as pltpu
from jax.experimental.pallas import tpu_sc as plsc

NUM_ITERS = 10
ALPHA = 0.1

NC = 2
NS = 16
CHUNK = 128
GROUP = 40


def _make_scatter_kernel(n_chunks_t, pn, ch):
    rows_per_tile = pn // NS
    n_zero = rows_per_tile // CHUNK
    n_groups = n_chunks_t // GROUP
    NBUF = 4
    n_quads = GROUP // NBUF

    @functools.partial(
        pl.kernel,
        out_type=jax.ShapeDtypeStruct((NC, pn, ch), jnp.float32),
        mesh=plsc.VectorSubcoreMesh(core_axis_name="c", subcore_axis_name="s",
                                    num_cores=NC),
        compiler_params=pltpu.CompilerParams(use_tc_tiling_on_sc=False),
        scratch_types=[
            pltpu.VMEM((GROUP, CHUNK), jnp.int32),
            pltpu.VMEM((GROUP, CHUNK), jnp.int32),
            [pltpu.VMEM((CHUNK, ch), jnp.float32)] * NBUF,
            pltpu.VMEM_SHARED((pn, ch), jnp.float32),
            [pltpu.SemaphoreType.DMA] * NBUF,
            [pltpu.SemaphoreType.DMA] * NBUF,
        ],
    )
    def scatter_kernel(src2_hbm, dst_hbm, y2_hbm, zeros_hbm, out_hbm,
                       src_idx, dst_idx, bufs, acc, semg, sems):
        cid = lax.axis_index("c")
        sid = lax.axis_index("s")
        base = sid * n_chunks_t

        pltpu.sync_copy(zeros_hbm, bufs[0])

        def zbody(r, carry):
            pltpu.sync_copy(
                bufs[0], acc.at[pl.ds(sid * rows_per_tile + r * CHUNK, CHUNK)])
            return carry

        lax.fori_loop(0, n_zero, zbody, 0, unroll=False)
        plsc.subcore_barrier()

        def gather(j, t):
            return pltpu.async_copy(y2_hbm.at[src_idx.at[j]], bufs[t], semg[t])

        def wait_gather(j, t):
            pltpu.make_async_copy(y2_hbm.at[src_idx.at[j]], bufs[t],
                                  semg[t]).wait()

        def scatter(j, t):
            return pltpu.async_copy(bufs[t], acc.at[dst_idx.at[j]], sems[t],
                                    add=True)

        def wait_scatter(j, t):
            pltpu.make_async_copy(bufs[t], acc.at[dst_idx.at[j]],
                                  sems[t]).wait()

        def group(g, carry):
            pltpu.sync_copy(
                src2_hbm.at[cid, pl.ds(base + g * GROUP, GROUP)], src_idx)
            pltpu.sync_copy(
                dst_hbm.at[pl.ds(base + g * GROUP, GROUP)], dst_idx)
            for t in range(NBUF):
                gather(t, t)

            def quad(q, carry2):
                for t in range(NBUF):
                    j = NBUF * q + t
                    wait_gather(j, t)

                    @pl.when(q > 0)
                    def _():
                        wait_scatter(j - NBUF, t)

                    scatter(j, t)

                    @pl.when(q < n_quads - 1)
                    def _():
                        gather(j + NBUF, t)

                return carry2

            lax.fori_loop(0, n_quads, quad, 0, unroll=False)
            for t in range(NBUF):
                wait_scatter(NBUF * (n_quads - 1) + t, t)
            return carry

        lax.fori_loop(0, n_groups, group, 0, unroll=False)
        plsc.subcore_barrier()

        def obody(r, carry):
            off = sid * rows_per_tile + r * CHUNK
            pltpu.sync_copy(acc.at[pl.ds(off, CHUNK)],
                            out_hbm.at[cid, pl.ds(off, CHUNK)])
            return carry

        lax.fori_loop(0, n_zero, obody, 0, unroll=False)

    return scatter_kernel


def _prep_body(w_ref, degp_ref, p_ref, norm_ref):
    w = w_ref[...]
    m = jnp.max(w, axis=1, keepdims=True)
    e = jnp.exp(w - m)
    p_ref[...] = e / jnp.sum(e, axis=1, keepdims=True)
    nrm = 1.0 / jnp.maximum(degp_ref[0], 1.0)
    norm_ref[...] = jnp.concatenate([nrm, nrm], axis=1)


def _y0_body(est_ref, p_ref, y_ref):
    y_ref[...] = jnp.dot(est_ref[...], p_ref[...],
                         preferred_element_type=jnp.float32)


def _blend_body(parts_ref, norm_ref, est0_ref, p_ref, blend_ref, y_ref):
    s = jnp.concatenate([parts_ref[0], parts_ref[1]], axis=1)
    b = (1.0 - ALPHA) * s * norm_ref[...] + ALPHA * est0_ref[...]
    blend_ref[...] = b
    y_ref[...] = jnp.dot(b, p_ref[...], preferred_element_type=jnp.float32)


def kernel(edge_index, estimates, W):
    n, c = estimates.shape
    ch = c // 2
    e = edge_index.shape[1]

    n_chunks_t = -(-(-(-e // (NS * CHUNK))) // GROUP) * GROUP
    e_pad = n_chunks_t * CHUNK * NS
    pn = -(-n // (NS * CHUNK)) * (NS * CHUNK)
    pad = e_pad - e
    pad_ar = jnp.arange(pad, dtype=jnp.int32)
    src = jnp.concatenate([edge_index[0], pad_ar % n]).reshape(-1, CHUNK)
    dst = jnp.concatenate([edge_index[1], n + pad_ar % (pn - n)]).reshape(-1, CHUNK)
    src2 = jnp.stack([2 * src, 2 * src + 1])

    zeros_h = jnp.zeros((CHUNK, ch), jnp.float32)
    ones_t = jnp.ones((2 * n, ch), jnp.float32)

    scatter = _make_scatter_kernel(n_chunks_t, pn, ch)

    deg_parts = scatter(src2, dst, ones_t, zeros_h)

    p_mat, norm = pl.pallas_call(
        _prep_body,
        out_shape=(jax.ShapeDtypeStruct((c, c), jnp.float32),
                   jax.ShapeDtypeStruct((pn, c), jnp.float32)),
    )(W, deg_parts)

    bn = 2000
    grid = n // bn
    y = pl.pallas_call(
        _y0_body,
        grid=(grid,),
        in_specs=[pl.BlockSpec((bn, c), lambda i: (i, 0)),
                  pl.BlockSpec((c, c), lambda i: (0, 0))],
        out_specs=pl.BlockSpec((bn, c), lambda i: (i, 0)),
        out_shape=jax.ShapeDtypeStruct((n, c), jnp.float32),
    )(estimates, p_mat)

    blend_call = pl.pallas_call(
        _blend_body,
        grid=(grid,),
        in_specs=[pl.BlockSpec((NC, bn, ch), lambda i: (0, i, 0)),
                  pl.BlockSpec((bn, c), lambda i: (i, 0)),
                  pl.BlockSpec((bn, c), lambda i: (i, 0)),
                  pl.BlockSpec((c, c), lambda i: (0, 0))],
        out_specs=(pl.BlockSpec((bn, c), lambda i: (i, 0)),
                   pl.BlockSpec((bn, c), lambda i: (i, 0))),
        out_shape=(jax.ShapeDtypeStruct((n, c), jnp.float32),
                   jax.ShapeDtypeStruct((n, c), jnp.float32)),
    )

    est = estimates
    for _ in range(NUM_ITERS):
        parts = scatter(src2, dst, y.reshape(2 * n, ch), zeros_h)
        est, y = blend_call(parts, norm, estimates, p_mat)
    return est
